# baseline (speedup 1.0000x reference)
import jax
import jax.numpy as jnp
from jax.experimental import pallas as pl

MIN_WINDOW_SIZE = 33
MAX_WINDOW_SIZE = 99

_BR = 512
_CW = 512


def _mask_kernel(boxes_ref, out_ref):
    r0 = pl.program_id(0) * _BR
    c0 = pl.program_id(1) * _CW

    on_band = (c0 + _CW > r0 - 49) & (c0 < r0 + _BR + 49)

    @pl.when(on_band)
    def _():
        wh = boxes_ref[:, 2:4]
        mx = jnp.max(wh, axis=1)
        mn = jnp.min(wh, axis=1)
        scale = jnp.sqrt(mx / mn)
        window = (MIN_WINDOW_SIZE * scale).astype(jnp.int32)
        window = jnp.clip(window, MIN_WINDOW_SIZE, MAX_WINDOW_SIZE)
        half = (window // 2).astype(jnp.int16)

        i = r0.astype(jnp.int16) + jax.lax.broadcasted_iota(
            jnp.int16, (_BR, _CW), 0)
        j = c0.astype(jnp.int16) + jax.lax.broadcasted_iota(
            jnp.int16, (_BR, _CW), 1)
        out_ref[...] = jnp.abs(j - i) <= half[:, None]

    @pl.when(jnp.logical_not(on_band))
    def _():
        out_ref[...] = jnp.zeros(out_ref.shape, jnp.bool_)


@jax.jit
def kernel(boxes):
    n = boxes.shape[0]
    grid = (pl.cdiv(n, _BR), pl.cdiv(n, _CW))
    return pl.pallas_call(
        _mask_kernel,
        grid=grid,
        in_specs=[pl.BlockSpec((_BR, 4), lambda r, c: (r, 0))],
        out_specs=pl.BlockSpec((_BR, _CW), lambda r, c: (r, c)),
        out_shape=jax.ShapeDtypeStruct((n, n), jnp.bool_),
    )(boxes)
